# baseline (speedup 1.0000x reference)
import functools

import jax
import jax.numpy as jnp
from jax import lax
from jax.experimental import pallas as pl
from jax.experimental.pallas import tpu as pltpu

BBLK = 256


def _cleanup_body(z_ref, cb_ref, out_ref, zf_buf, sem):
    bblk, f_total, d = z_ref.shape
    _, k_total, _ = cb_ref.shape

    def slice_copy(f, slot):
        return pltpu.make_async_copy(
            z_ref.at[:, f, :], zf_buf.at[slot], sem.at[slot]
        )

    slice_copy(0, 0).start()
    acc = None
    for f in range(f_total):
        if f + 1 < f_total:
            slice_copy(f + 1, (f + 1) % 2).start()
        slice_copy(f, f % 2).wait()
        zf = zf_buf[f % 2]
        cbf = cb_ref[f]
        sims = lax.dot_general(
            zf, cbf, (((1,), (1,)), ((), ())),
            preferred_element_type=jnp.float32,
        )
        idx = jnp.argmax(jnp.abs(sims), axis=1)
        onehot = (
            idx[:, None] == lax.broadcasted_iota(jnp.int32, (bblk, k_total), 1)
        ).astype(jnp.bfloat16)
        wf = lax.dot_general(
            onehot, cbf.astype(jnp.bfloat16), (((1,), (0,)), ((), ())),
            preferred_element_type=jnp.float32,
        )
        acc = wf if acc is None else acc * wf
    out_ref[...] = acc


@jax.jit
def kernel(z, codebooks):
    b, f, d = z.shape
    return pl.pallas_call(
        _cleanup_body,
        grid=(b // BBLK,),
        in_specs=[
            pl.BlockSpec((BBLK, f, d), lambda i: (i, 0, 0)),
            pl.BlockSpec(codebooks.shape, lambda i: (0, 0, 0)),
        ],
        out_specs=pl.BlockSpec((BBLK, d), lambda i: (i, 0)),
        out_shape=jax.ShapeDtypeStruct((b, d), jnp.float32),
        scratch_shapes=[
            pltpu.VMEM((2, BBLK, d), jnp.float32),
            pltpu.SemaphoreType.DMA((2,)),
        ],
        compiler_params=pltpu.CompilerParams(
            dimension_semantics=("arbitrary",),
        ),
    )(z, codebooks)


# speedup vs baseline: 1.5706x; 1.5706x over previous
import functools

import jax
import jax.numpy as jnp
from jax import lax
from jax.experimental import pallas as pl
from jax.experimental.pallas import tpu as pltpu

BBLK = 256


def _cleanup_body(z_hbm, cb_ref, out_ref, zbuf, sem):
    i = pl.program_id(0)
    f = pl.program_id(1)
    n_i = pl.num_programs(0)
    f_total = pl.num_programs(1)
    _, k_total, d = cb_ref.shape

    def slice_copy(bi, fi, slot):
        return pltpu.make_async_copy(
            z_hbm.at[pl.ds(bi * BBLK, BBLK), fi, :],
            zbuf.at[slot],
            sem.at[slot],
        )

    @pl.when(jnp.logical_and(i == 0, f == 0))
    def _prologue():
        slice_copy(0, 0, 0).start()

    @pl.when(jnp.logical_or(i + 1 < n_i, f + 1 < f_total))
    def _prefetch():
        nxt_i = jnp.where(f + 1 < f_total, i, i + 1)
        nxt_f = jnp.where(f + 1 < f_total, f + 1, 0)
        slice_copy(nxt_i, nxt_f, (f + 1) % 2).start()

    slice_copy(i, f, f % 2).wait()
    zf = zbuf[f % 2]
    cbf = cb_ref[f]
    sims = lax.dot_general(
        zf, cbf, (((1,), (1,)), ((), ())),
        preferred_element_type=jnp.float32,
    )
    idx = jnp.argmax(jnp.abs(sims), axis=1)
    onehot = (
        idx[:, None] == lax.broadcasted_iota(jnp.int32, (BBLK, k_total), 1)
    ).astype(jnp.bfloat16)
    wf = lax.dot_general(
        onehot, cbf.astype(jnp.bfloat16), (((1,), (0,)), ((), ())),
        preferred_element_type=jnp.float32,
    )

    @pl.when(f == 0)
    def _init():
        out_ref[...] = wf

    @pl.when(f != 0)
    def _acc():
        out_ref[...] = out_ref[...] * wf


@jax.jit
def kernel(z, codebooks):
    b, f, d = z.shape
    return pl.pallas_call(
        _cleanup_body,
        grid=(b // BBLK, f),
        in_specs=[
            pl.BlockSpec(memory_space=pl.ANY),
            pl.BlockSpec(codebooks.shape, lambda i, j: (0, 0, 0)),
        ],
        out_specs=pl.BlockSpec((BBLK, d), lambda i, j: (i, 0)),
        out_shape=jax.ShapeDtypeStruct((b, d), jnp.float32),
        scratch_shapes=[
            pltpu.VMEM((2, BBLK, d), jnp.float32),
            pltpu.SemaphoreType.DMA((2,)),
        ],
        compiler_params=pltpu.CompilerParams(
            dimension_semantics=("parallel", "arbitrary"),
        ),
    )(z, codebooks)


# speedup vs baseline: 1.7754x; 1.1304x over previous
import functools

import jax
import jax.numpy as jnp
from jax import lax
from jax.experimental import pallas as pl
from jax.experimental.pallas import tpu as pltpu

BBLK = 256


def _cleanup_body(z_hbm, cb_ref, out_ref, zbuf, sem):
    i = pl.program_id(0)
    f = pl.program_id(1)
    n_i = pl.num_programs(0)
    f_total = pl.num_programs(1)
    _, k_total, d = cb_ref.shape

    def slice_copy(bi, fi, slot):
        return pltpu.make_async_copy(
            z_hbm.at[pl.ds(bi * BBLK, BBLK), fi, :],
            zbuf.at[slot],
            sem.at[slot],
        )

    t = i * f_total + f

    @pl.when(t == 0)
    def _prologue():
        slice_copy(0, 0, 0).start()
        slice_copy(0, 1, 1).start()
        slice_copy(0, 2, 2).start()

    @pl.when(t + 3 < n_i * f_total)
    def _prefetch():
        nxt = t + 3
        slice_copy(nxt // f_total, nxt % f_total, nxt % 4).start()

    slice_copy(i, f, f % 4).wait()
    zf = zbuf[f % 4]
    cbf = cb_ref[f]
    sims = lax.dot_general(
        zf, cbf, (((1,), (1,)), ((), ())),
        preferred_element_type=jnp.float32,
    )
    idx = jnp.argmax(jnp.abs(sims), axis=1)
    onehot = (
        idx[:, None] == lax.broadcasted_iota(jnp.int32, (BBLK, k_total), 1)
    ).astype(jnp.bfloat16)
    wf = lax.dot_general(
        onehot, cbf.astype(jnp.bfloat16), (((1,), (0,)), ((), ())),
        preferred_element_type=jnp.float32,
    )

    @pl.when(f == 0)
    def _init():
        out_ref[...] = wf

    @pl.when(f != 0)
    def _acc():
        out_ref[...] = out_ref[...] * wf


@jax.jit
def kernel(z, codebooks):
    b, f, d = z.shape
    return pl.pallas_call(
        _cleanup_body,
        grid=(b // BBLK, f),
        in_specs=[
            pl.BlockSpec(memory_space=pl.ANY),
            pl.BlockSpec(codebooks.shape, lambda i, j: (0, 0, 0)),
        ],
        out_specs=pl.BlockSpec((BBLK, d), lambda i, j: (i, 0)),
        out_shape=jax.ShapeDtypeStruct((b, d), jnp.float32),
        scratch_shapes=[
            pltpu.VMEM((4, BBLK, d), jnp.float32),
            pltpu.SemaphoreType.DMA((4,)),
        ],
        compiler_params=pltpu.CompilerParams(
            dimension_semantics=("parallel", "arbitrary"),
        ),
    )(z, codebooks)


# speedup vs baseline: 2.1476x; 1.2096x over previous
import functools

import jax
import jax.numpy as jnp
from jax import lax
from jax.experimental import pallas as pl
from jax.experimental.pallas import tpu as pltpu

BBLK = 256


def _cleanup_body(z_ref, cb_ref, cbh_ref, out_ref):
    bblk, f_total, d = z_ref.shape
    _, k_total, _ = cb_ref.shape
    slices = [z_ref[:, f, :] for f in range(f_total)]
    acc = None
    for f in range(f_total):
        sims = lax.dot_general(
            slices[f], cb_ref[f], (((1,), (1,)), ((), ())),
            preferred_element_type=jnp.float32,
        )
        idx = jnp.argmax(jnp.abs(sims), axis=1)
        onehot = (
            idx[:, None] == lax.broadcasted_iota(jnp.int32, (bblk, k_total), 1)
        ).astype(jnp.bfloat16)
        wf = lax.dot_general(
            onehot, cbh_ref[f], (((1,), (0,)), ((), ())),
            preferred_element_type=jnp.float32,
        )
        acc = wf if acc is None else acc * wf
    out_ref[...] = acc


@jax.jit
def kernel(z, codebooks):
    b, f, d = z.shape
    return pl.pallas_call(
        _cleanup_body,
        grid=(b // BBLK,),
        in_specs=[
            pl.BlockSpec((BBLK, f, d), lambda i: (i, 0, 0)),
            pl.BlockSpec(codebooks.shape, lambda i: (0, 0, 0)),
            pl.BlockSpec(codebooks.shape, lambda i: (0, 0, 0)),
        ],
        out_specs=pl.BlockSpec((BBLK, d), lambda i: (i, 0)),
        out_shape=jax.ShapeDtypeStruct((b, d), jnp.float32),
        compiler_params=pltpu.CompilerParams(
            dimension_semantics=("arbitrary",),
        ),
    )(z, codebooks, codebooks.astype(jnp.bfloat16))
